# baseline (speedup 1.0000x reference)
import jax
import jax.numpy as jnp
from jax.experimental import pallas as pl
from jax.experimental.pallas import tpu as pltpu

_TEMPERATURE = 0.25
_THRESHOLD = 0.5


def _body(q_ref, wq_ref, wr_ref, keys_ref, vals_ref, out_ref):
    q = q_ref[0]
    qe = jax.lax.dot_general(q, wq_ref[...], (((1,), (1,)), ((), ())),
                             preferred_element_type=jnp.float32)
    e = jnp.exp(qe * (1.0 / _TEMPERATURE))
    denom = 1.0 + jnp.sum(e, axis=-1, keepdims=True)
    num = jnp.concatenate([e, jnp.ones_like(denom)], axis=-1)
    qs = jnp.sqrt(num / denom)
    scores = jax.lax.dot_general(qs, keys_ref[...], (((1,), (1,)), ((), ())),
                                 preferred_element_type=jnp.float32)
    s2 = scores * scores
    s4 = s2 * s2
    s8 = s4 * s4
    mx = jnp.max(s8, axis=-1, keepdims=True)
    thr = jnp.where(mx < _THRESHOLD, 0.9 * mx, _THRESHOLD)
    masked = jnp.where(s8 < thr, 0.0, s8)
    w = masked / (jnp.sum(masked, axis=-1, keepdims=True) + 1e-9)
    read = jax.lax.dot_general(w, vals_ref[0], (((1,), (0,)), ((), ())),
                               preferred_element_type=jnp.float32)
    out_ref[0] = jax.lax.dot_general(read, wr_ref[...], (((1,), (1,)), ((), ())),
                                     preferred_element_type=jnp.float32)


def kernel(queries, W_query, W_read, memory_keys, memory_values):
    B, Q, QD = queries.shape
    VD, ED = W_read.shape
    M, EDp1 = memory_keys.shape
    return pl.pallas_call(
        _body,
        grid=(B,),
        in_specs=[
            pl.BlockSpec((1, Q, QD), lambda b: (b, 0, 0)),
            pl.BlockSpec((ED, QD), lambda b: (0, 0)),
            pl.BlockSpec((VD, ED), lambda b: (0, 0)),
            pl.BlockSpec((M, EDp1), lambda b: (0, 0)),
            pl.BlockSpec((1, M, ED), lambda b: (b, 0, 0)),
        ],
        out_specs=pl.BlockSpec((1, Q, VD), lambda b: (b, 0, 0)),
        out_shape=jax.ShapeDtypeStruct((B, Q, VD), jnp.float32),
        compiler_params=pltpu.CompilerParams(
            dimension_semantics=("arbitrary",)),
    )(queries, W_query, W_read, memory_keys, memory_values)
